# Initial kernel scaffold; baseline (speedup 1.0000x reference)
import jax
import jax.numpy as jnp
from jax import lax
from jax.experimental import pallas as pl
from jax.experimental.pallas import tpu as pltpu
from jax.experimental.pallas import tpu_sc as plsc

SIZE = 1383
P = SIZE * SIZE
W = 2 * P
N_SIG = 1024 * 1024

CB = 16384
PH = CB // 2
NFULL = W // CB
TAIL_W = W - NFULL * CB
TAIL_P = TAIL_W // 2

NW = 16
ROWS_TOTAL = N_SIG // 128
ROWS_PER_W = ROWS_TOTAL // NW
SUB = 4
ROWS_PER_SUB = ROWS_PER_W // SUB


def _body(phi2_hbm, val_hbm, idx_hbm, out_hbm, a_ref, b_ref, idxv, valv, sem):
    wid = lax.axis_index("s")
    iota16 = lax.iota(jnp.int32, 16)
    iota2 = iota16 * 2 + 1
    zeros16 = jnp.zeros((16,), jnp.float32)

    def zbody(i, c):
        b_ref[pl.ds(i * 16, 16)] = zeros16
        return c
    lax.fori_loop(0, CB // 16, zbody, 0, unroll=8)

    def fill_chunk(ci, c):
        chunk = wid + ci * NW
        @pl.when(chunk < NFULL)
        def _():
            pltpu.sync_copy(phi2_hbm.at[pl.ds(chunk * PH, PH)], a_ref)
            def body(i, c2):
                v = a_ref[pl.ds(i * 16, 16)]
                plsc.store_scatter(b_ref, [iota2 + i * 32], v)
                return c2
            lax.fori_loop(0, PH // 16, body, 0, unroll=8)
            pltpu.sync_copy(b_ref, out_hbm.at[pl.ds(chunk * CB, CB)])
        return c
    lax.fori_loop(0, (NFULL + NW - 1) // NW, fill_chunk, 0)

    @pl.when(wid == 0)
    def _():
        pltpu.sync_copy(phi2_hbm.at[pl.ds(NFULL * PH, TAIL_P)],
                        a_ref.at[pl.ds(0, TAIL_P)])
        def tbody(i, c2):
            v = a_ref[pl.ds(i * 16, 16)]
            m = (iota16 + i * 16) < TAIL_P
            plsc.store_scatter(b_ref, [iota2 + i * 32], v, mask=m)
            return c2
        lax.fori_loop(0, (TAIL_P + 15) // 16, tbody, 0, unroll=8)
        pltpu.sync_copy(b_ref.at[pl.ds(0, TAIL_W)],
                        out_hbm.at[pl.ds(NFULL * CB, TAIL_W)])

    plsc.subcore_barrier()

    base_row = wid * ROWS_PER_W
    def sub_chunk(s, c):
        r0 = base_row + s * ROWS_PER_SUB
        pltpu.sync_copy(idx_hbm.at[pl.ds(r0, ROWS_PER_SUB)], idxv)
        pltpu.sync_copy(val_hbm.at[pl.ds(r0, ROWS_PER_SUB)], valv)
        def fire_group(g, c2):
            for bq in range(16):
                j = g * 16 + bq
                pltpu.make_async_copy(valv.at[j], out_hbm.at[idxv.at[j]],
                                      sem).start()
            for bq in range(16):
                j = g * 16 + bq
                pltpu.make_async_copy(valv.at[j], out_hbm.at[idxv.at[j]],
                                      sem).wait()
            return c2
        lax.fori_loop(0, ROWS_PER_SUB // 16, fire_group, 0)
        return c
    lax.fori_loop(0, SUB, sub_chunk, 0)


def kernel(x, phi2, koordinates):
    phi2f = phi2.reshape(-1)
    xf = x.reshape(ROWS_TOTAL, 128)
    idx2 = (koordinates[:N_SIG, 0].astype(jnp.int32) * 2).reshape(ROWS_TOTAL, 128)

    mesh = plsc.VectorSubcoreMesh(core_axis_name="c", subcore_axis_name="s",
                                  num_cores=1)
    out = pl.kernel(
        _body,
        out_type=jax.ShapeDtypeStruct((W,), jnp.float32),
        mesh=mesh,
        scratch_types=[
            pltpu.VMEM((PH,), jnp.float32),
            pltpu.VMEM((CB,), jnp.float32),
            pltpu.VMEM((ROWS_PER_SUB, 128), jnp.int32),
            pltpu.VMEM((ROWS_PER_SUB, 128), jnp.float32),
            pltpu.SemaphoreType.DMA,
        ],
    )(phi2f, xf, idx2)
    return out.reshape(1, SIZE, SIZE, 2)


# speedup vs baseline: 1.5019x; 1.5019x over previous
import jax
import jax.numpy as jnp
from jax import lax
from jax.experimental import pallas as pl
from jax.experimental.pallas import tpu as pltpu
from jax.experimental.pallas import tpu_sc as plsc

SIZE = 1383
P = SIZE * SIZE
W = 2 * P
N_SIG = 1024 * 1024

CB = 16384
PH = CB // 2
NFULL = W // CB
TAIL_W = W - NFULL * CB
TAIL_P = TAIL_W // 2

NW = 16
ROWS_TOTAL = N_SIG // 128
ROWS_PER_W = ROWS_TOTAL // NW
SUB = 4
ROWS_PER_SUB = ROWS_PER_W // SUB


def _body(phi2_hbm, val_hbm, idx_hbm, out_hbm, a_ref, b_ref, idxv, valv, sem):
    wid = lax.axis_index("s")
    iota16 = lax.iota(jnp.int32, 16)
    iota2 = iota16 * 2 + 1
    zeros16 = jnp.zeros((16,), jnp.float32)

    def zbody(i, c):
        b_ref[pl.ds(i * 16, 16)] = zeros16
        return c
    lax.fori_loop(0, CB // 16, zbody, 0, unroll=8)

    def fill_chunk(ci, c):
        chunk = wid + ci * NW
        @pl.when(chunk < NFULL)
        def _():
            pltpu.sync_copy(phi2_hbm.at[pl.ds(chunk * PH, PH)], a_ref)
            def body(i, c2):
                v = a_ref[pl.ds(i * 16, 16)]
                plsc.store_scatter(b_ref, [iota2 + i * 32], v)
                return c2
            lax.fori_loop(0, PH // 16, body, 0, unroll=8)
            pltpu.sync_copy(b_ref, out_hbm.at[pl.ds(chunk * CB, CB)])
        return c
    lax.fori_loop(0, (NFULL + NW - 1) // NW, fill_chunk, 0)

    @pl.when(wid == 0)
    def _():
        pltpu.sync_copy(phi2_hbm.at[pl.ds(NFULL * PH, TAIL_P)],
                        a_ref.at[pl.ds(0, TAIL_P)])
        def tbody(i, c2):
            v = a_ref[pl.ds(i * 16, 16)]
            m = (iota16 + i * 16) < TAIL_P
            plsc.store_scatter(b_ref, [iota2 + i * 32], v, mask=m)
            return c2
        lax.fori_loop(0, (TAIL_P + 15) // 16, tbody, 0, unroll=8)
        pltpu.sync_copy(b_ref.at[pl.ds(0, TAIL_W)],
                        out_hbm.at[pl.ds(NFULL * CB, TAIL_W)])

    plsc.subcore_barrier()

    base_row = wid * ROWS_PER_W
    def sub_chunk(s, c):
        r0 = base_row + s * ROWS_PER_SUB
        pltpu.sync_copy(idx_hbm.at[pl.ds(r0, ROWS_PER_SUB)], idxv)
        pltpu.sync_copy(val_hbm.at[pl.ds(r0, ROWS_PER_SUB)], valv)
        def fire_group(g, c2):
            for bq in range(16):
                j = g * 16 + bq
                pltpu.make_async_copy(valv.at[j], out_hbm.at[idxv.at[j]],
                                      sem).start()
            for bq in range(16):
                j = g * 16 + bq
                pltpu.make_async_copy(valv.at[j], out_hbm.at[idxv.at[j]],
                                      sem).wait()
            return c2
        lax.fori_loop(0, ROWS_PER_SUB // 16, fire_group, 0)
        return c
    lax.fori_loop(0, SUB, sub_chunk, 0)


def kernel(x, phi2, koordinates):
    phi2f = phi2.reshape(-1)
    xf = x.reshape(ROWS_TOTAL, 128)
    idx2 = (koordinates[:N_SIG, 0].astype(jnp.int32) * 2).reshape(ROWS_TOTAL, 128)

    mesh = plsc.VectorSubcoreMesh(core_axis_name="c", subcore_axis_name="s",
                                  num_cores=1, num_subcores=NW)
    out = pl.kernel(
        _body,
        out_type=jax.ShapeDtypeStruct((W,), jnp.float32),
        mesh=mesh,
        compiler_params=pltpu.CompilerParams(needs_layout_passes=False),
        scratch_types=[
            pltpu.VMEM((PH,), jnp.float32),
            pltpu.VMEM((CB,), jnp.float32),
            pltpu.VMEM((ROWS_PER_SUB, 128), jnp.int32),
            pltpu.VMEM((ROWS_PER_SUB, 128), jnp.float32),
            pltpu.SemaphoreType.DMA,
        ],
    )(phi2f, xf, idx2)
    return out.reshape(1, SIZE, SIZE, 2)


# speedup vs baseline: 1.5052x; 1.0022x over previous
import jax
import jax.numpy as jnp
from jax import lax
from jax.experimental import pallas as pl
from jax.experimental.pallas import tpu as pltpu
from jax.experimental.pallas import tpu_sc as plsc

SIZE = 1383
P = SIZE * SIZE
W = 2 * P
N_SIG = 1024 * 1024

CB = 16384
PH = CB // 2
NFULL = W // CB
TAIL_W = W - NFULL * CB
TAIL_P = TAIL_W // 2

NW = 16
ELEMS_PER_W = N_SIG // NW
SUB = 4
SCHUNK = ELEMS_PER_W // SUB

FILL_ITERS = (NFULL + NW - 1) // NW


def _body(phi2_hbm, val_hbm, idx_hbm, out_hbm,
          a0, a1, b0, b1, idx0, idx1, val0, val1, lA, lB, sA, sB):
    wid = lax.axis_index("s")
    iota16 = lax.iota(jnp.int32, 16)
    iota2 = iota16 * 2 + 1
    zeros16 = jnp.zeros((16,), jnp.float32)
    a_bufs, b_bufs = (a0, a1), (b0, b1)
    idx_bufs, val_bufs = (idx0, idx1), (val0, val1)
    lsems, ssems = (lA, lB), (sA, sB)

    def zbody(i, c):
        b0[pl.ds(i * 16, 16)] = zeros16
        b1[pl.ds(i * 16, 16)] = zeros16
        return c
    lax.fori_loop(0, CB // 16, zbody, 0, unroll=8)

    def load_desc(ci, buf):
        c = wid + ci * NW
        return pltpu.make_async_copy(
            phi2_hbm.at[pl.ds(c * PH, PH)], a_bufs[buf], lsems[buf])

    def store_desc(ci, buf):
        c = wid + ci * NW
        return pltpu.make_async_copy(
            b_bufs[buf], out_hbm.at[pl.ds(c * CB, CB)], ssems[buf])

    def guard(ci, fn):
        c = wid + ci * NW
        @pl.when(c < NFULL)
        def _():
            fn()

    guard(0, lambda: load_desc(0, 0).start())
    for ci in range(FILL_ITERS):
        cb = ci % 2
        if ci + 1 < FILL_ITERS:
            guard(ci + 1, lambda ci=ci: load_desc(ci + 1, (ci + 1) % 2).start())
        if ci >= 2:
            guard(ci - 2, lambda ci=ci, cb=cb: store_desc(ci - 2, cb).wait())
        guard(ci, lambda ci=ci, cb=cb: load_desc(ci, cb).wait())

        def compute(cb=cb):
            a_ref, b_ref = a_bufs[cb], b_bufs[cb]
            def body(i, c2):
                v = a_ref[pl.ds(i * 16, 16)]
                plsc.store_scatter(b_ref, [iota2 + i * 32], v)
                return c2
            lax.fori_loop(0, PH // 16, body, 0, unroll=4)
        guard(ci, compute)
        guard(ci, lambda ci=ci, cb=cb: store_desc(ci, cb).start())
    for ci in (FILL_ITERS - 2, FILL_ITERS - 1):
        guard(ci, lambda ci=ci: store_desc(ci, ci % 2).wait())

    @pl.when(wid == 0)
    def _():
        def rz(i, c):
            b0[pl.ds(i * 16, 16)] = zeros16
            return c
        lax.fori_loop(0, TAIL_W // 16 + 1, rz, 0, unroll=8)
        pltpu.sync_copy(phi2_hbm.at[pl.ds(NFULL * PH, TAIL_P)],
                        a0.at[pl.ds(0, TAIL_P)])
        def tbody(i, c2):
            v = a0[pl.ds(i * 16, 16)]
            m = (iota16 + i * 16) < TAIL_P
            plsc.store_scatter(b0, [iota2 + i * 32], v, mask=m)
            return c2
        lax.fori_loop(0, (TAIL_P + 15) // 16, tbody, 0, unroll=4)
        pltpu.sync_copy(b0.at[pl.ds(0, TAIL_W)],
                        out_hbm.at[pl.ds(NFULL * CB, TAIL_W)])

    plsc.subcore_barrier()

    base = wid * ELEMS_PER_W

    def ld_descs(s, buf):
        e0 = base + s * SCHUNK
        return (pltpu.make_async_copy(idx_hbm.at[pl.ds(e0, SCHUNK)],
                                      idx_bufs[buf], lsems[buf]),
                pltpu.make_async_copy(val_hbm.at[pl.ds(e0, SCHUNK)],
                                      val_bufs[buf], lsems[buf]))

    def sc_desc(buf):
        return pltpu.make_async_copy(val_bufs[buf], out_hbm.at[idx_bufs[buf]],
                                     ssems[buf])

    for de in ld_descs(0, 0):
        de.start()
    for s in range(SUB):
        sb = s % 2
        if s + 1 < SUB:
            if s >= 1:
                sc_desc((s + 1) % 2).wait()
            for de in ld_descs(s + 1, (s + 1) % 2):
                de.start()
        for de in ld_descs(s, sb):
            de.wait()
        sc_desc(sb).start()
    sc_desc((SUB - 2) % 2).wait()
    sc_desc((SUB - 1) % 2).wait()


def kernel(x, phi2, koordinates):
    phi2f = phi2.reshape(-1)
    xf = x.reshape(-1)
    idx2 = koordinates[:N_SIG, 0].astype(jnp.int32) * 2

    mesh = plsc.VectorSubcoreMesh(core_axis_name="c", subcore_axis_name="s",
                                  num_cores=1, num_subcores=NW)
    out = pl.kernel(
        _body,
        out_type=jax.ShapeDtypeStruct((W,), jnp.float32),
        mesh=mesh,
        compiler_params=pltpu.CompilerParams(needs_layout_passes=False),
        scratch_types=[
            pltpu.VMEM((PH,), jnp.float32),
            pltpu.VMEM((PH,), jnp.float32),
            pltpu.VMEM((CB,), jnp.float32),
            pltpu.VMEM((CB,), jnp.float32),
            pltpu.VMEM((SCHUNK,), jnp.int32),
            pltpu.VMEM((SCHUNK,), jnp.int32),
            pltpu.VMEM((SCHUNK,), jnp.float32),
            pltpu.VMEM((SCHUNK,), jnp.float32),
            pltpu.SemaphoreType.DMA,
            pltpu.SemaphoreType.DMA,
            pltpu.SemaphoreType.DMA,
            pltpu.SemaphoreType.DMA,
        ],
    )(phi2f, xf, idx2)
    return out.reshape(1, SIZE, SIZE, 2)


# speedup vs baseline: 2.4897x; 1.6541x over previous
import jax
import jax.numpy as jnp
from jax import lax
from jax.experimental import pallas as pl
from jax.experimental.pallas import tpu as pltpu
from jax.experimental.pallas import tpu_sc as plsc

SIZE = 1383
P = SIZE * SIZE
N_SIG = 1024 * 1024
NT = 11
ROWW = NT * 256
WPHYS = SIZE * ROWW

RB = 8
NBLK = SIZE // RB
TAIL_R = SIZE - NBLK * RB
A_LEN = RB * SIZE
B_LEN = RB * ROWW

NW = 16
ELEMS_PER_W = N_SIG // NW
SUB = 8
SCHUNK = ELEMS_PER_W // SUB

BLK_ITERS = (NBLK + NW - 1) // NW


def _body(phi2_hbm, val_hbm, idx_hbm, out_hbm,
          a0, a1, b0, b1, idx0, idx1, val0, val1, lA, lB, sA, sB):
    wid = lax.axis_index("s")
    iota16 = lax.iota(jnp.int32, 16)
    zeros16 = jnp.zeros((16,), jnp.float32)
    a_bufs, b_bufs = (a0, a1), (b0, b1)
    idx_bufs, val_bufs = (idx0, idx1), (val0, val1)
    lsems, ssems = (lA, lB), (sA, sB)

    def zbody(i, c):
        b0[pl.ds(i * 16, 16)] = zeros16
        b1[pl.ds(i * 16, 16)] = zeros16
        return c
    lax.fori_loop(0, B_LEN // 16, zbody, 0, unroll=8)

    def load_desc(ci, buf):
        c = wid + ci * NW
        return pltpu.make_async_copy(
            phi2_hbm.at[pl.ds(c * A_LEN, A_LEN)],
            a_bufs[buf].at[pl.ds(0, A_LEN)], lsems[buf])

    def store_desc(ci, buf):
        c = wid + ci * NW
        return pltpu.make_async_copy(
            b_bufs[buf], out_hbm.at[pl.ds(c * B_LEN, B_LEN)], ssems[buf])

    def copy_rows(a_ref, b_ref, nrows):
        def row(r, c2):
            asrc = r * SIZE
            bdst = r * ROWW
            def tile(jt, c3):
                for k in range(8):
                    v = plsc.load_gather(
                        a_ref, [asrc + jt * 128 + k * 16 + iota16])
                    b_ref[pl.ds(bdst + jt * 256 + 128 + k * 16, 16)] = v
                return c3
            lax.fori_loop(0, NT - 1, tile, 0)
            for k in range(6):
                v = plsc.load_gather(a_ref, [asrc + 1280 + k * 16 + iota16])
                b_ref[pl.ds(bdst + 2688 + k * 16, 16)] = v
            v = plsc.load_gather(a_ref, [asrc + 1376 + iota16])
            plsc.store_scatter(b_ref, [bdst + 2784 + iota16], v,
                               mask=iota16 < 7)
            return c2
        lax.fori_loop(0, nrows, row, 0)

    def fill_slot(ci, t):
        cur_ok = (wid + ci * NW) < NBLK
        nxt_ok = (wid + (ci + 1) * NW) < NBLK
        prv_ok = jnp.logical_and(ci >= 2, (wid + (ci - 2) * NW) < NBLK)
        @pl.when(jnp.logical_and(ci + 1 <= BLK_ITERS - 1, nxt_ok))
        def _():
            load_desc(ci + 1, 1 - t).start()
        @pl.when(prv_ok)
        def _():
            store_desc(ci - 2, t).wait()
        @pl.when(cur_ok)
        def _():
            load_desc(ci, t).wait()
            copy_rows(a_bufs[t], b_bufs[t], RB)
            store_desc(ci, t).start()

    @pl.when(wid < NBLK)
    def _():
        load_desc(0, 0).start()
    def fill_pair(q, c):
        fill_slot(2 * q, 0)
        fill_slot(2 * q + 1, 1)
        return c
    lax.fori_loop(0, (BLK_ITERS + 2) // 2, fill_pair, 0)
    @pl.when((wid + (BLK_ITERS - 1) * NW) < NBLK)
    def _():
        store_desc(BLK_ITERS - 1, (BLK_ITERS - 1) % 2).wait()

    @pl.when(wid == NW - 1)
    def _():
        pltpu.sync_copy(phi2_hbm.at[pl.ds(NBLK * A_LEN, TAIL_R * SIZE)],
                        a1.at[pl.ds(0, TAIL_R * SIZE)])
        copy_rows(a1, b1, TAIL_R)
        pltpu.sync_copy(b1.at[pl.ds(0, TAIL_R * ROWW)],
                        out_hbm.at[pl.ds(NBLK * B_LEN, TAIL_R * ROWW)])

    plsc.subcore_barrier()

    base = wid * ELEMS_PER_W

    def ld_descs(s, buf):
        e0 = base + s * SCHUNK
        return (pltpu.make_async_copy(idx_hbm.at[pl.ds(e0, SCHUNK)],
                                      idx_bufs[buf], lsems[buf]),
                pltpu.make_async_copy(val_hbm.at[pl.ds(e0, SCHUNK)],
                                      val_bufs[buf], lsems[buf]))

    def sc_desc(buf):
        return pltpu.make_async_copy(val_bufs[buf], out_hbm.at[idx_bufs[buf]],
                                     ssems[buf])

    def sc_slot(s, t):
        @pl.when(jnp.logical_and(s >= 1, s + 1 < SUB))
        def _():
            sc_desc(1 - t).wait()
        @pl.when(s + 1 < SUB)
        def _():
            for de in ld_descs(s + 1, 1 - t):
                de.start()
        @pl.when(s < SUB)
        def _():
            for de in ld_descs(s, t):
                de.wait()
            sc_desc(t).start()

    for de in ld_descs(0, 0):
        de.start()
    def sc_pair(q, c):
        sc_slot(2 * q, 0)
        sc_slot(2 * q + 1, 1)
        return c
    lax.fori_loop(0, SUB // 2, sc_pair, 0)
    sc_desc((SUB - 2) % 2).wait()
    sc_desc((SUB - 1) % 2).wait()


def kernel(x, phi2, koordinates):
    phi2f = phi2.reshape(-1)
    xf = x.reshape(-1)
    p = koordinates[:N_SIG, 0].astype(jnp.int32)
    i = p // SIZE
    j = p - i * SIZE
    idxp = i * ROWW + (j // 128) * 256 + (j % 128)

    mesh = plsc.VectorSubcoreMesh(core_axis_name="c", subcore_axis_name="s",
                                  num_cores=1, num_subcores=NW)
    out = pl.kernel(
        _body,
        out_type=jax.ShapeDtypeStruct((WPHYS,), jnp.float32),
        mesh=mesh,
        compiler_params=pltpu.CompilerParams(needs_layout_passes=False),
        scratch_types=[
            pltpu.VMEM((A_LEN + 16,), jnp.float32),
            pltpu.VMEM((A_LEN + 16,), jnp.float32),
            pltpu.VMEM((B_LEN,), jnp.float32),
            pltpu.VMEM((B_LEN,), jnp.float32),
            pltpu.VMEM((SCHUNK,), jnp.int32),
            pltpu.VMEM((SCHUNK,), jnp.int32),
            pltpu.VMEM((SCHUNK,), jnp.float32),
            pltpu.VMEM((SCHUNK,), jnp.float32),
            pltpu.SemaphoreType.DMA,
            pltpu.SemaphoreType.DMA,
            pltpu.SemaphoreType.DMA,
            pltpu.SemaphoreType.DMA,
        ],
    )(phi2f, xf, idxp)
    a4 = out.reshape(SIZE, NT, 2, 128)
    full = a4.transpose(0, 1, 3, 2).reshape(SIZE, NT * 128, 2)
    return full[:, :SIZE, :].reshape(1, SIZE, SIZE, 2)


# speedup vs baseline: 11.2999x; 4.5387x over previous
import jax
import jax.numpy as jnp
from jax import lax
from jax.experimental import pallas as pl
from jax.experimental.pallas import tpu as pltpu
from jax.experimental.pallas import tpu_sc as plsc

SIZE = 1383
P = SIZE * SIZE
N_SIG = 1024 * 1024
NT = 11
ROWW = NT * 256
WPHYS = SIZE * ROWW

SSTR = SIZE + 1
H1_ROWS = 696
H2_ROWS = SIZE - H1_ROWS
H1_SP = H1_ROWS * SSTR
H2_SP = H2_ROWS * SSTR
DUMP1 = H1_SP
DUMP2 = H2_SP
DSPREAD = 2048
SP_ALLOC = H1_SP + DSPREAD

RB = 8
A_LEN = RB * SIZE
C_LEN = RB * SSTR
B_LEN = RB * ROWW
N1BLK = H1_ROWS // RB
N2BLK = H2_ROWS // RB
TAIL_R = H2_ROWS - N2BLK * RB

NZF = SP_ALLOC // B_LEN
ZTAIL = SP_ALLOC - NZF * B_LEN

NW = 16
ELEMS_PER_W = N_SIG // NW
SUB = 16
SCHUNK = ELEMS_PER_W // SUB


def _body(phi2_hbm, val_hbm, idxa_hbm, idxb_hbm, out_hbm,
          a_ref, c_ref, b_ref, idx0, idx1, val0, val1,
          raster, lA, lB, sA, sB):
    wid = lax.axis_index("s")
    iota16 = lax.iota(jnp.int32, 16)
    zeros16 = jnp.zeros((16,), jnp.float32)
    idx_bufs, val_bufs = (idx0, idx1), (val0, val1)
    lsems, ssems = (lA, lB), (sA, sB)

    def zbody(i, c):
        b_ref[pl.ds(i * 16, 16)] = zeros16
        return c
    lax.fori_loop(0, B_LEN // 16, zbody, 0, unroll=8)

    def zero_raster():
        def zc(k, c):
            ch = wid + k * NW
            @pl.when(ch < NZF)
            def _():
                pltpu.sync_copy(b_ref, raster.at[pl.ds(ch * B_LEN, B_LEN)])
            return c
        lax.fori_loop(0, (NZF + NW - 1) // NW, zc, 0)
        @pl.when(wid == 11)
        def _():
            pltpu.sync_copy(b_ref.at[pl.ds(0, ZTAIL)],
                            raster.at[pl.ds(NZF * B_LEN, ZTAIL)])

    def scatter_pass(idx_hbm):
        base = wid * ELEMS_PER_W

        def ld_descs(s, buf):
            e0 = base + s * SCHUNK
            return (pltpu.make_async_copy(idx_hbm.at[pl.ds(e0, SCHUNK)],
                                          idx_bufs[buf], lsems[buf]),
                    pltpu.make_async_copy(val_hbm.at[pl.ds(e0, SCHUNK)],
                                          val_bufs[buf], lsems[buf]))

        def sc_desc(buf):
            return pltpu.make_async_copy(val_bufs[buf],
                                         raster.at[idx_bufs[buf]], ssems[buf])

        def sc_slot(s, t):
            @pl.when(jnp.logical_and(s >= 1, s + 1 < SUB))
            def _():
                sc_desc(1 - t).wait()
            @pl.when(s + 1 < SUB)
            def _():
                for de in ld_descs(s + 1, 1 - t):
                    de.start()
            @pl.when(s < SUB)
            def _():
                for de in ld_descs(s, t):
                    de.wait()
                sc_desc(t).start()

        for de in ld_descs(0, 0):
            de.start()
        def sc_pair(q, c):
            sc_slot(2 * q, 0)
            sc_slot(2 * q + 1, 1)
            return c
        lax.fori_loop(0, SUB // 2, sc_pair, 0)
        sc_desc((SUB - 2) % 2).wait()
        sc_desc((SUB - 1) % 2).wait()

    def copy_rows(nrows):
        def row(r, c2):
            asrc = r * SIZE
            ssrc = r * SSTR
            bdst = r * ROWW
            def tile(jt, c3):
                for k in range(8):
                    sv = plsc.load_gather(
                        c_ref, [ssrc + jt * 128 + k * 16 + iota16])
                    b_ref[pl.ds(bdst + jt * 256 + k * 16, 16)] = sv
                    pv = plsc.load_gather(
                        a_ref, [asrc + jt * 128 + k * 16 + iota16])
                    b_ref[pl.ds(bdst + jt * 256 + 128 + k * 16, 16)] = pv
                return c3
            lax.fori_loop(0, NT - 1, tile, 0)
            for k in range(6):
                sv = plsc.load_gather(c_ref, [ssrc + 1280 + k * 16 + iota16])
                b_ref[pl.ds(bdst + 2560 + k * 16, 16)] = sv
                pv = plsc.load_gather(a_ref, [asrc + 1280 + k * 16 + iota16])
                b_ref[pl.ds(bdst + 2688 + k * 16, 16)] = pv
            m7 = iota16 < 7
            sv = plsc.load_gather(c_ref, [ssrc + 1376 + iota16])
            plsc.store_scatter(b_ref, [bdst + 2656 + iota16], sv, mask=m7)
            pv = plsc.load_gather(a_ref, [asrc + 1376 + iota16])
            plsc.store_scatter(b_ref, [bdst + 2784 + iota16], pv, mask=m7)
            return c2
        lax.fori_loop(0, nrows, row, 0)

    def merge_pass(nblk, row_base):
        def mslot(k, c):
            blk = wid + k * NW
            @pl.when(blk < nblk)
            def _():
                g0 = row_base + blk * RB
                pltpu.sync_copy(
                    phi2_hbm.at[pl.ds(g0 * SIZE, A_LEN)],
                    a_ref.at[pl.ds(0, A_LEN)])
                pltpu.sync_copy(
                    raster.at[pl.ds(blk * RB * SSTR, C_LEN)],
                    c_ref.at[pl.ds(0, C_LEN)])
                copy_rows(RB)
                pltpu.sync_copy(
                    b_ref, out_hbm.at[pl.ds(g0 * ROWW, B_LEN)])
            return c
        lax.fori_loop(0, (nblk + NW - 1) // NW, mslot, 0)

    zero_raster()
    plsc.subcore_barrier()
    scatter_pass(idxa_hbm)
    plsc.subcore_barrier()
    merge_pass(N1BLK, 0)
    plsc.subcore_barrier()

    lax.fori_loop(0, B_LEN // 16, zbody, 0, unroll=8)
    zero_raster()
    plsc.subcore_barrier()
    scatter_pass(idxb_hbm)
    plsc.subcore_barrier()
    merge_pass(N2BLK, H1_ROWS)

    @pl.when(wid == NW - 1)
    def _():
        pltpu.sync_copy(phi2_hbm.at[pl.ds(1376 * SIZE, TAIL_R * SIZE)],
                        a_ref.at[pl.ds(0, TAIL_R * SIZE)])
        pltpu.sync_copy(raster.at[pl.ds(N2BLK * RB * SSTR, TAIL_R * SSTR)],
                        c_ref.at[pl.ds(0, TAIL_R * SSTR)])
        copy_rows(TAIL_R)
        pltpu.sync_copy(b_ref.at[pl.ds(0, TAIL_R * ROWW)],
                        out_hbm.at[pl.ds(1376 * ROWW, TAIL_R * ROWW)])


def kernel(x, phi2, koordinates):
    phi2f = phi2.reshape(-1)
    xf = x.reshape(-1)
    p = koordinates[:N_SIG, 0].astype(jnp.int32)
    row = p // SIZE
    col = p - row * SIZE
    spread = jnp.bitwise_and(p, DSPREAD - 1)
    idxa = jnp.where(row < H1_ROWS, row * SSTR + col, DUMP1 + spread)
    idxb = jnp.where(row >= H1_ROWS, (row - H1_ROWS) * SSTR + col,
                     DUMP2 + spread)

    mesh = plsc.VectorSubcoreMesh(core_axis_name="c", subcore_axis_name="s",
                                  num_cores=1, num_subcores=NW)
    out = pl.kernel(
        _body,
        out_type=jax.ShapeDtypeStruct((WPHYS,), jnp.float32),
        mesh=mesh,
        compiler_params=pltpu.CompilerParams(needs_layout_passes=False),
        scratch_types=[
            pltpu.VMEM((A_LEN + 16,), jnp.float32),
            pltpu.VMEM((C_LEN + 16,), jnp.float32),
            pltpu.VMEM((B_LEN,), jnp.float32),
            pltpu.VMEM((SCHUNK,), jnp.int32),
            pltpu.VMEM((SCHUNK,), jnp.int32),
            pltpu.VMEM((SCHUNK,), jnp.float32),
            pltpu.VMEM((SCHUNK,), jnp.float32),
            pltpu.VMEM_SHARED((SP_ALLOC,), jnp.float32),
            pltpu.SemaphoreType.DMA,
            pltpu.SemaphoreType.DMA,
            pltpu.SemaphoreType.DMA,
            pltpu.SemaphoreType.DMA,
        ],
    )(phi2f, xf, idxa, idxb)
    a4 = out.reshape(SIZE, NT, 2, 128)
    full = a4.transpose(0, 1, 3, 2).reshape(SIZE, NT * 128, 2)
    return full[:, :SIZE, :].reshape(1, SIZE, SIZE, 2)


# speedup vs baseline: 11.7541x; 1.0402x over previous
import jax
import jax.numpy as jnp
from jax import lax
from jax.experimental import pallas as pl
from jax.experimental.pallas import tpu as pltpu
from jax.experimental.pallas import tpu_sc as plsc

SIZE = 1383
P = SIZE * SIZE
N_SIG = 1024 * 1024
NT = 11
ROWW = NT * 256
WPHYS = SIZE * ROWW

SSTR = SIZE + 1
H1_ROWS = 696
H2_ROWS = SIZE - H1_ROWS
H1_SP = H1_ROWS * SSTR
H2_SP = H2_ROWS * SSTR
DUMP1 = H1_SP
DUMP2 = H2_SP
DSPREAD = 2048
SP_ALLOC = H1_SP + DSPREAD

RB = 8
A_LEN = RB * SIZE
C_LEN = RB * SSTR
B_LEN = RB * ROWW
N1BLK = H1_ROWS // RB
N2BLK = H2_ROWS // RB
TAIL_R = H2_ROWS - N2BLK * RB

NZF = SP_ALLOC // B_LEN
ZTAIL = SP_ALLOC - NZF * B_LEN

NW = 16
ELEMS_PER_W = N_SIG // NW
SUB = 16
SCHUNK = ELEMS_PER_W // SUB


def _body(phi2_hbm, val_hbm, idxa_hbm, idxb_hbm, out_hbm,
          a_ref, c_ref, b_ref, idx0, idx1, val0, val1,
          raster, lA, lB, sA, sB):
    wid = lax.axis_index("s")
    iota16 = lax.iota(jnp.int32, 16)
    zeros16 = jnp.zeros((16,), jnp.float32)
    idx_bufs, val_bufs = (idx0, idx1), (val0, val1)
    lsems, ssems = (lA, lB), (sA, sB)

    def zbody(i, c):
        b_ref[pl.ds(i * 16, 16)] = zeros16
        return c
    lax.fori_loop(0, B_LEN // 16, zbody, 0, unroll=8)

    def zero_raster():
        def zc(k, c):
            ch = wid + k * NW
            @pl.when(ch < NZF)
            def _():
                pltpu.sync_copy(b_ref, raster.at[pl.ds(ch * B_LEN, B_LEN)])
            return c
        lax.fori_loop(0, (NZF + NW - 1) // NW, zc, 0)
        @pl.when(wid == 11)
        def _():
            pltpu.sync_copy(b_ref.at[pl.ds(0, ZTAIL)],
                            raster.at[pl.ds(NZF * B_LEN, ZTAIL)])

    def scatter_pass(idx_hbm):
        base = wid * ELEMS_PER_W

        def ld_descs(s, buf):
            e0 = base + s * SCHUNK
            return (pltpu.make_async_copy(idx_hbm.at[pl.ds(e0, SCHUNK)],
                                          idx_bufs[buf], lsems[buf]),
                    pltpu.make_async_copy(val_hbm.at[pl.ds(e0, SCHUNK)],
                                          val_bufs[buf], lsems[buf]))

        def sc_desc(buf):
            return pltpu.make_async_copy(val_bufs[buf],
                                         raster.at[idx_bufs[buf]], ssems[buf])

        def sc_slot(s, t):
            @pl.when(jnp.logical_and(s >= 1, s + 1 < SUB))
            def _():
                sc_desc(1 - t).wait()
            @pl.when(s + 1 < SUB)
            def _():
                for de in ld_descs(s + 1, 1 - t):
                    de.start()
            @pl.when(s < SUB)
            def _():
                for de in ld_descs(s, t):
                    de.wait()
                sc_desc(t).start()

        for de in ld_descs(0, 0):
            de.start()
        def sc_pair(q, c):
            sc_slot(2 * q, 0)
            sc_slot(2 * q + 1, 1)
            return c
        lax.fori_loop(0, SUB // 2, sc_pair, 0)
        sc_desc((SUB - 2) % 2).wait()
        sc_desc((SUB - 1) % 2).wait()

    def copy_rows(nrows):
        def row(r, c2):
            ssrc = r * SSTR
            bdst = r * ROWW
            def tile(jt, c3):
                for k in range(8):
                    sv = plsc.load_gather(
                        c_ref, [ssrc + jt * 128 + k * 16 + iota16])
                    b_ref[pl.ds(bdst + jt * 256 + k * 16, 16)] = sv
                    pv = a_ref[jt, r, pl.ds(k * 16, 16)]
                    b_ref[pl.ds(bdst + jt * 256 + 128 + k * 16, 16)] = pv
                return c3
            lax.fori_loop(0, NT - 1, tile, 0)
            for k in range(6):
                sv = plsc.load_gather(c_ref, [ssrc + 1280 + k * 16 + iota16])
                b_ref[pl.ds(bdst + 2560 + k * 16, 16)] = sv
                pv = a_ref[10, r, pl.ds(k * 16, 16)]
                b_ref[pl.ds(bdst + 2688 + k * 16, 16)] = pv
            m7 = iota16 < 7
            sv = plsc.load_gather(c_ref, [ssrc + 1376 + iota16])
            plsc.store_scatter(b_ref, [bdst + 2656 + iota16], sv, mask=m7)
            pv = a_ref[10, r, pl.ds(96, 16)]
            plsc.store_scatter(b_ref, [bdst + 2784 + iota16], pv, mask=m7)
            return c2
        lax.fori_loop(0, nrows, row, 0)

    def merge_pass(nblk, row_base):
        def mslot(k, c):
            blk = wid + k * NW
            @pl.when(blk < nblk)
            def _():
                g0 = row_base + blk * RB
                pltpu.sync_copy(phi2_hbm.at[row_base // RB + blk], a_ref)
                pltpu.sync_copy(
                    raster.at[pl.ds(blk * RB * SSTR, C_LEN)],
                    c_ref.at[pl.ds(0, C_LEN)])
                copy_rows(RB)
                pltpu.sync_copy(
                    b_ref, out_hbm.at[pl.ds(g0 * ROWW, B_LEN)])
            return c
        lax.fori_loop(0, (nblk + NW - 1) // NW, mslot, 0)

    zero_raster()
    plsc.subcore_barrier()
    scatter_pass(idxa_hbm)
    plsc.subcore_barrier()
    merge_pass(N1BLK, 0)
    plsc.subcore_barrier()

    lax.fori_loop(0, B_LEN // 16, zbody, 0, unroll=8)
    zero_raster()
    plsc.subcore_barrier()
    scatter_pass(idxb_hbm)
    plsc.subcore_barrier()
    merge_pass(N2BLK, H1_ROWS)

    @pl.when(wid == NW - 1)
    def _():
        pltpu.sync_copy(phi2_hbm.at[172], a_ref)
        pltpu.sync_copy(raster.at[pl.ds(N2BLK * RB * SSTR, TAIL_R * SSTR)],
                        c_ref.at[pl.ds(0, TAIL_R * SSTR)])
        copy_rows(TAIL_R)
        pltpu.sync_copy(b_ref.at[pl.ds(0, TAIL_R * ROWW)],
                        out_hbm.at[pl.ds(1376 * ROWW, TAIL_R * ROWW)])


def kernel(x, phi2, koordinates):
    phi2t = jnp.pad(phi2, ((0, 1), (0, 25))).reshape(173, 8, 11, 128)
    phi2t = phi2t.transpose(0, 2, 1, 3)
    xf = x.reshape(-1)
    p = koordinates[:N_SIG, 0].astype(jnp.int32)
    row = p // SIZE
    col = p - row * SIZE
    spread = jnp.bitwise_and(p, DSPREAD - 1)
    idxa = jnp.where(row < H1_ROWS, row * SSTR + col, DUMP1 + spread)
    idxb = jnp.where(row >= H1_ROWS, (row - H1_ROWS) * SSTR + col,
                     DUMP2 + spread)

    mesh = plsc.VectorSubcoreMesh(core_axis_name="c", subcore_axis_name="s",
                                  num_cores=1, num_subcores=NW)
    out = pl.kernel(
        _body,
        out_type=jax.ShapeDtypeStruct((WPHYS,), jnp.float32),
        mesh=mesh,
        compiler_params=pltpu.CompilerParams(needs_layout_passes=False),
        scratch_types=[
            pltpu.VMEM((11, 8, 128), jnp.float32),
            pltpu.VMEM((C_LEN + 16,), jnp.float32),
            pltpu.VMEM((B_LEN,), jnp.float32),
            pltpu.VMEM((SCHUNK,), jnp.int32),
            pltpu.VMEM((SCHUNK,), jnp.int32),
            pltpu.VMEM((SCHUNK,), jnp.float32),
            pltpu.VMEM((SCHUNK,), jnp.float32),
            pltpu.VMEM_SHARED((SP_ALLOC,), jnp.float32),
            pltpu.SemaphoreType.DMA,
            pltpu.SemaphoreType.DMA,
            pltpu.SemaphoreType.DMA,
            pltpu.SemaphoreType.DMA,
        ],
    )(phi2t, xf, idxa, idxb)
    a4 = out.reshape(SIZE, NT, 2, 128)
    full = a4.transpose(0, 1, 3, 2).reshape(SIZE, NT * 128, 2)
    return full[:, :SIZE, :].reshape(1, SIZE, SIZE, 2)


# speedup vs baseline: 11.7576x; 1.0003x over previous
import jax
import jax.numpy as jnp
from jax import lax
from jax.experimental import pallas as pl
from jax.experimental.pallas import tpu as pltpu
from jax.experimental.pallas import tpu_sc as plsc
from jax.experimental import layout as jxl

SIZE = 1383
P = SIZE * SIZE
N_SIG = 1024 * 1024
NT = 11
ROWW = NT * 256
WPHYS = SIZE * ROWW

SSTR = SIZE + 1
H1_ROWS = 696
H2_ROWS = SIZE - H1_ROWS
H1_SP = H1_ROWS * SSTR
H2_SP = H2_ROWS * SSTR
DUMP1 = H1_SP
DUMP2 = H2_SP
DSPREAD = 2048
SP_ALLOC = H1_SP + DSPREAD

RB = 8
A_LEN = RB * SIZE
C_LEN = RB * SSTR
B_LEN = RB * ROWW
N1BLK = H1_ROWS // RB
N2BLK = H2_ROWS // RB
TAIL_R = H2_ROWS - N2BLK * RB

NZF = SP_ALLOC // B_LEN
ZTAIL = SP_ALLOC - NZF * B_LEN

NW = 16
ELEMS_PER_W = N_SIG // NW
SUB = 16
SCHUNK = ELEMS_PER_W // SUB


def _body(phi2_hbm, val_hbm, idxa_hbm, idxb_hbm, out_hbm,
          a_ref, c_ref, b_ref, idx0, idx1, val0, val1,
          raster, lA, lB, sA, sB):
    wid = lax.axis_index("s")
    iota16 = lax.iota(jnp.int32, 16)
    zeros16 = jnp.zeros((16,), jnp.float32)
    idx_bufs, val_bufs = (idx0, idx1), (val0, val1)
    lsems, ssems = (lA, lB), (sA, sB)

    def zbody(i, c):
        b_ref[pl.ds(i * 16, 16)] = zeros16
        return c
    lax.fori_loop(0, B_LEN // 16, zbody, 0, unroll=8)

    def zero_raster():
        def zc(k, c):
            ch = wid + k * NW
            @pl.when(ch < NZF)
            def _():
                pltpu.sync_copy(b_ref, raster.at[pl.ds(ch * B_LEN, B_LEN)])
            return c
        lax.fori_loop(0, (NZF + NW - 1) // NW, zc, 0)
        @pl.when(wid == 11)
        def _():
            pltpu.sync_copy(b_ref.at[pl.ds(0, ZTAIL)],
                            raster.at[pl.ds(NZF * B_LEN, ZTAIL)])

    def scatter_pass(idx_hbm):
        base = wid * ELEMS_PER_W

        def ld_descs(s, buf):
            e0 = base + s * SCHUNK
            return (pltpu.make_async_copy(idx_hbm.at[pl.ds(e0, SCHUNK)],
                                          idx_bufs[buf], lsems[buf]),
                    pltpu.make_async_copy(val_hbm.at[pl.ds(e0, SCHUNK)],
                                          val_bufs[buf], lsems[buf]))

        def sc_desc(buf):
            return pltpu.make_async_copy(val_bufs[buf],
                                         raster.at[idx_bufs[buf]], ssems[buf])

        def sc_slot(s, t):
            @pl.when(jnp.logical_and(s >= 1, s + 1 < SUB))
            def _():
                sc_desc(1 - t).wait()
            @pl.when(s + 1 < SUB)
            def _():
                for de in ld_descs(s + 1, 1 - t):
                    de.start()
            @pl.when(s < SUB)
            def _():
                for de in ld_descs(s, t):
                    de.wait()
                sc_desc(t).start()

        for de in ld_descs(0, 0):
            de.start()
        def sc_pair(q, c):
            sc_slot(2 * q, 0)
            sc_slot(2 * q + 1, 1)
            return c
        lax.fori_loop(0, SUB // 2, sc_pair, 0)
        sc_desc((SUB - 2) % 2).wait()
        sc_desc((SUB - 1) % 2).wait()

    def copy_rows(nrows):
        def row(r, c2):
            ssrc = r * SSTR
            bdst = r * ROWW
            def tile(jt, c3):
                for k in range(8):
                    sv = plsc.load_gather(
                        c_ref, [ssrc + jt * 128 + k * 16 + iota16])
                    b_ref[pl.ds(bdst + jt * 256 + k * 16, 16)] = sv
                    pv = a_ref[jt, r, pl.ds(k * 16, 16)]
                    b_ref[pl.ds(bdst + jt * 256 + 128 + k * 16, 16)] = pv
                return c3
            lax.fori_loop(0, NT - 1, tile, 0)
            for k in range(6):
                sv = plsc.load_gather(c_ref, [ssrc + 1280 + k * 16 + iota16])
                b_ref[pl.ds(bdst + 2560 + k * 16, 16)] = sv
                pv = a_ref[10, r, pl.ds(k * 16, 16)]
                b_ref[pl.ds(bdst + 2688 + k * 16, 16)] = pv
            m7 = iota16 < 7
            sv = plsc.load_gather(c_ref, [ssrc + 1376 + iota16])
            plsc.store_scatter(b_ref, [bdst + 2656 + iota16], sv, mask=m7)
            pv = a_ref[10, r, pl.ds(96, 16)]
            plsc.store_scatter(b_ref, [bdst + 2784 + iota16], pv, mask=m7)
            return c2
        lax.fori_loop(0, nrows, row, 0)

    def merge_pass(nblk, row_base):
        def mslot(k, c):
            blk = wid + k * NW
            @pl.when(blk < nblk)
            def _():
                g0 = row_base + blk * RB
                pltpu.sync_copy(phi2_hbm.at[row_base // RB + blk], a_ref)
                pltpu.sync_copy(
                    raster.at[pl.ds(blk * RB * SSTR, C_LEN)],
                    c_ref.at[pl.ds(0, C_LEN)])
                copy_rows(RB)
                pltpu.sync_copy(
                    b_ref, out_hbm.at[pl.ds(g0 * ROWW, B_LEN)])
            return c
        lax.fori_loop(0, (nblk + NW - 1) // NW, mslot, 0)

    zero_raster()
    plsc.subcore_barrier()
    scatter_pass(idxa_hbm)
    plsc.subcore_barrier()
    merge_pass(N1BLK, 0)
    plsc.subcore_barrier()

    lax.fori_loop(0, B_LEN // 16, zbody, 0, unroll=8)
    zero_raster()
    plsc.subcore_barrier()
    scatter_pass(idxb_hbm)
    plsc.subcore_barrier()
    merge_pass(N2BLK, H1_ROWS)

    @pl.when(wid == NW - 1)
    def _():
        pltpu.sync_copy(phi2_hbm.at[172], a_ref)
        pltpu.sync_copy(raster.at[pl.ds(N2BLK * RB * SSTR, TAIL_R * SSTR)],
                        c_ref.at[pl.ds(0, TAIL_R * SSTR)])
        copy_rows(TAIL_R)
        pltpu.sync_copy(b_ref.at[pl.ds(0, TAIL_R * ROWW)],
                        out_hbm.at[pl.ds(1376 * ROWW, TAIL_R * ROWW)])


def kernel(x, phi2, koordinates):
    phi2t = jnp.pad(phi2, ((0, 1), (0, 25))).reshape(173, 8, 11, 128)
    phi2t = phi2t.transpose(0, 2, 1, 3)
    xf = x.reshape(-1)
    p = koordinates[:N_SIG, 0].astype(jnp.int32)
    row = p // SIZE
    col = p - row * SIZE
    spread = jnp.bitwise_and(p, DSPREAD - 1)
    idxa = jnp.where(row < H1_ROWS, row * SSTR + col, DUMP1 + spread)
    idxb = jnp.where(row >= H1_ROWS, (row - H1_ROWS) * SSTR + col,
                     DUMP2 + spread)

    mesh = plsc.VectorSubcoreMesh(core_axis_name="c", subcore_axis_name="s",
                                  num_cores=1, num_subcores=NW)
    out = pl.kernel(
        _body,
        out_type=jax.ShapeDtypeStruct((WPHYS,), jnp.float32),
        mesh=mesh,
        compiler_params=pltpu.CompilerParams(needs_layout_passes=False),
        scratch_types=[
            pltpu.VMEM((11, 8, 128), jnp.float32),
            pltpu.VMEM((C_LEN + 16,), jnp.float32),
            pltpu.VMEM((B_LEN,), jnp.float32),
            pltpu.VMEM((SCHUNK,), jnp.int32),
            pltpu.VMEM((SCHUNK,), jnp.int32),
            pltpu.VMEM((SCHUNK,), jnp.float32),
            pltpu.VMEM((SCHUNK,), jnp.float32),
            pltpu.VMEM_SHARED((SP_ALLOC,), jnp.float32),
            pltpu.SemaphoreType.DMA,
            pltpu.SemaphoreType.DMA,
            pltpu.SemaphoreType.DMA,
            pltpu.SemaphoreType.DMA,
        ],
    )(phi2t, xf, idxa, idxb)
    a4 = out.reshape(SIZE, NT, 2, 128)
    full = a4.transpose(0, 1, 3, 2).reshape(SIZE, NT * 128, 2)
    full = jxl.with_layout_constraint(
        full, jxl.Layout(major_to_minor=(0, 2, 1), tiling=((2, 128),)))
    return full[:, :SIZE, :].reshape(1, SIZE, SIZE, 2)
